# baseline (speedup 1.0000x reference)
import functools

import jax
import jax.numpy as jnp
from jax import lax
from jax.experimental import pallas as pl
from jax.experimental.pallas import tpu as pltpu
from jax.experimental.pallas import tpu_sc as plsc

N = 10000
D = 128
NUM_CLASSES = 32
MAX_SEQ_LEN = 16
BATCH_SIZE = 64

BLK = 1024
GRID = 10
NPAD = BLK * GRID
DEH = 80
SCOL = 48
ACCR = 10112
PAD_DST = N

NSUB = 16
CHUNK = 128
CPT = 162
EPT = CPT * CHUNK
ETOT = NSUB * EPT
RPT = ACCR // NSUB
PACK = 16384


def _layer0_tc(x_ref, w_ref, as_ref, ad_ref, hh_ref, es_ref, ed_ref, mx_ref):
    i = pl.program_id(0)
    xb = x_ref[...]
    h = jnp.dot(xb, w_ref[...], preferred_element_type=jnp.float32)
    _finish_layer_tc(i, h, as_ref, ad_ref, hh_ref, es_ref, ed_ref, mx_ref)


def _finish_layer_tc(i, h, as_ref, ad_ref, hh_ref, es_ref, ed_ref, mx_ref):
    lane = jax.lax.broadcasted_iota(jnp.int32, (BLK, 32), 1)
    extra = jnp.where(lane == 0, 1.0, 0.0).astype(jnp.float32)
    hh_ref[0] = h[:, :DEH]
    hh_ref[1] = jnp.concatenate([h[:, DEH:128], extra], axis=1)
    es = jnp.sum(h * as_ref[...], axis=1, keepdims=True)
    ed = jnp.sum(h * ad_ref[...], axis=1, keepdims=True)
    es_ref[...] = es
    ed_ref[...] = ed
    m = jnp.broadcast_to(jnp.max(es), (1, 128))

    @pl.when(i == 0)
    def _():
        mx_ref[...] = m

    @pl.when(i > 0)
    def _():
        mx_ref[...] = jnp.maximum(mx_ref[...], m)


def _layer1_tc(acc_ref, b0_ref, w_ref, as_ref, ad_ref,
               hh_ref, es_ref, ed_ref, mx_ref):
    i = pl.program_id(0)
    al = acc_ref[0]
    ar = acc_ref[1]
    s = ar[:, SCOL:SCOL + 1]
    x1 = jnp.concatenate([al, ar[:, :SCOL]], axis=1) / s + b0_ref[...]
    x1 = jnp.maximum(x1, 0.0)
    row = jax.lax.broadcasted_iota(jnp.int32, (BLK, 1), 0) + i * BLK
    x1 = jnp.where(row < N, x1, 0.0)
    h = jnp.dot(x1, w_ref[...], preferred_element_type=jnp.float32)
    _finish_layer_tc(i, h, as_ref, ad_ref, hh_ref, es_ref, ed_ref, mx_ref)


def _head_gather_tc(n0_ref, acc_ref, b1_ref, out_ref):
    al = acc_ref[0, 0]
    ar = acc_ref[1, 0]
    s = ar[:, SCOL:SCOL + 1]
    out_ref[0] = jnp.concatenate([al, ar[:, :SCOL]], axis=1) / s + b1_ref[...]


def _head_mm_tc(h_ref, wp_ref, bp_ref, out_ref):
    out_ref[...] = (jnp.dot(h_ref[...], wp_ref[...],
                            preferred_element_type=jnp.float32) + bp_ref[...])


def _run_layer_tc(kern, first_arg, w, a_s, a_d, extra_b=None):
    ins = []
    in_specs = []
    if extra_b is None:
        in_specs.append(pl.BlockSpec((BLK, 128), lambda i: (i, 0)))
        ins.append(first_arg)
    else:
        in_specs.append(pl.BlockSpec((2, BLK, DEH), lambda i: (0, i, 0)))
        ins.append(first_arg)
        in_specs.append(pl.BlockSpec((1, 128), lambda i: (0, 0)))
        ins.append(extra_b.reshape(1, 128))
    in_specs += [
        pl.BlockSpec((128, 128), lambda i: (0, 0)),
        pl.BlockSpec((1, 128), lambda i: (0, 0)),
        pl.BlockSpec((1, 128), lambda i: (0, 0)),
    ]
    ins += [w, a_s.reshape(1, 128), a_d.reshape(1, 128)]
    return pl.pallas_call(
        kern,
        grid=(GRID,),
        in_specs=in_specs,
        out_specs=[
            pl.BlockSpec((2, BLK, DEH), lambda i: (0, i, 0)),
            pl.BlockSpec((BLK, 1), lambda i: (i, 0)),
            pl.BlockSpec((BLK, 1), lambda i: (i, 0)),
            pl.BlockSpec((1, 128), lambda i: (0, 0)),
        ],
        out_shape=[
            jax.ShapeDtypeStruct((2, NPAD, DEH), jnp.float32),
            jax.ShapeDtypeStruct((NPAD, 1), jnp.float32),
            jax.ShapeDtypeStruct((NPAD, 1), jnp.float32),
            jax.ShapeDtypeStruct((1, 128), jnp.float32),
        ],
    )(*ins)


def _make_sc_segment():
    mesh = plsc.VectorSubcoreMesh(core_axis_name="c", subcore_axis_name="s")

    @functools.partial(
        pl.kernel,
        mesh=mesh,
        out_type=jax.ShapeDtypeStruct((2, ACCR, DEH), jnp.float32),
        scratch_types=[
            pltpu.VMEM((NPAD,), jnp.float32),
            pltpu.VMEM((NPAD,), jnp.float32),
            pltpu.VMEM((128,), jnp.float32),
            pltpu.VMEM((CPT, CHUNK), jnp.int32),
            pltpu.VMEM((CHUNK,), jnp.int32),
            pltpu.VMEM((CHUNK,), jnp.int32),
            pltpu.VMEM((CHUNK,), jnp.float32),
            pltpu.VMEM((CHUNK, DEH), jnp.float32),
            pltpu.VMEM((CHUNK,), jnp.int32),
            pltpu.VMEM((CHUNK,), jnp.int32),
            pltpu.VMEM((CHUNK,), jnp.float32),
            pltpu.VMEM((CHUNK, DEH), jnp.float32),
            pltpu.VMEM_SHARED((ACCR, DEH), jnp.float32),
            pltpu.SemaphoreType.DMA,
            pltpu.SemaphoreType.DMA,
            pltpu.SemaphoreType.DMA,
            pltpu.SemaphoreType.DMA,
        ],
        compiler_params=pltpu.CompilerParams(needs_layout_passes=False,
                                             use_tc_tiling_on_sc=False),
    )
    def seg(hh_hbm, es_hbm, ed_hbm, mx_hbm, eidx_hbm, out_hbm,
            es_v, ed_v, mx_v, eidx_v,
            s_a, d_a, w_a, r_a, s_b, d_b, w_b, r_b,
            acc_sh, gsem_a, gsem_b, asem_a, asem_b):
        c = lax.axis_index("c")
        s = lax.axis_index("s")

        pltpu.sync_copy(es_hbm, es_v)
        pltpu.sync_copy(ed_hbm, ed_v)
        pltpu.sync_copy(mx_hbm, mx_v)
        pltpu.sync_copy(eidx_hbm.at[s], eidx_v)
        m16 = mx_v[pl.ds(0, 16)]

        def _zrow(r, carry):
            for j in range(DEH // 16):
                r_a[r, pl.ds(16 * j, 16)] = jnp.zeros((16,), jnp.float32)
                r_b[r, pl.ds(16 * j, 16)] = jnp.zeros((16,), jnp.float32)
            return carry
        lax.fori_loop(0, CHUNK, _zrow, 0)
        for v in range(CHUNK // 16):
            d_a[pl.ds(16 * v, 16)] = jnp.zeros((16,), jnp.int32)
            d_b[pl.ds(16 * v, 16)] = jnp.zeros((16,), jnp.int32)
        for off, sz in ((0, 128), (128, 128), (256, 128), (384, 128),
                        (512, 120)):
            pltpu.sync_copy(r_a.at[pl.ds(0, sz)],
                            acc_sh.at[pl.ds(s * RPT + off, sz)])
        plsc.subcore_barrier()

        pltpu.async_copy(r_a, acc_sh.at[d_a], asem_a, add=True)
        pltpu.async_copy(r_b, acc_sh.at[d_b], asem_b, add=True)

        def _weights(ci, s_x, d_x, w_x):
            def _wvec(v, c2):
                pk = eidx_v[ci, pl.ds(16 * v, 16)]
                si = jax.lax.shift_right_logical(pk, 14)
                di = jax.lax.bitwise_and(pk, PACK - 1)
                s_x[pl.ds(16 * v, 16)] = si
                d_x[pl.ds(16 * v, 16)] = di
                es_g = plsc.load_gather(es_v, [si])
                ed_g = plsc.load_gather(ed_v, [di])
                e = es_g + ed_g
                e = jnp.where(e > 0.0, e, 0.2 * e)
                w_x[pl.ds(16 * v, 16)] = jnp.exp(
                    e - jnp.maximum(ed_g + m16, 0.0))
                return c2
            lax.fori_loop(0, CHUNK // 16, _wvec, 0)

        def _scale(r_x, w_x):
            def _svec(v, c2):
                for l in range(16):
                    r = 16 * v + l
                    ws = plsc.load_gather(w_x, [jnp.full((16,), r, jnp.int32)])
                    for j in range(DEH // 16):
                        r_x[r, pl.ds(16 * j, 16)] = (
                            r_x[r, pl.ds(16 * j, 16)] * ws)
                return c2
            lax.fori_loop(0, CHUNK // 16, _svec, 0)

        def _pair(ci2, carry):
            pltpu.make_async_copy(r_a, acc_sh.at[d_a], asem_a).wait()
            _weights(2 * ci2, s_a, d_a, w_a)
            g_a = pltpu.async_copy(hh_hbm.at[c].at[s_a], r_a, gsem_a)

            pltpu.make_async_copy(r_b, acc_sh.at[d_b], asem_b).wait()
            _weights(2 * ci2 + 1, s_b, d_b, w_b)
            g_b = pltpu.async_copy(hh_hbm.at[c].at[s_b], r_b, gsem_b)

            g_a.wait()
            _scale(r_a, w_a)
            pltpu.async_copy(r_a, acc_sh.at[d_a], asem_a, add=True)

            g_b.wait()
            _scale(r_b, w_b)
            pltpu.async_copy(r_b, acc_sh.at[d_b], asem_b, add=True)
            return carry
        lax.fori_loop(0, CPT // 2, _pair, 0)

        pltpu.make_async_copy(r_a, acc_sh.at[d_a], asem_a).wait()
        pltpu.make_async_copy(r_b, acc_sh.at[d_b], asem_b).wait()
        plsc.subcore_barrier()
        pltpu.sync_copy(acc_sh.at[pl.ds(s * RPT, RPT)],
                        out_hbm.at[c, pl.ds(s * RPT, RPT)])

    return seg


_sc_segment = _make_sc_segment()


def _segment_sc(hh, esf, edf, mx, eidx):
    return _sc_segment(hh, esf, edf, mx, eidx)


def kernel(x, edge_index, batch, W0, a_src0, a_dst0, b0,
           W1, a_src1, a_dst1, b1, Wp, bp):
    loop = jnp.arange(N, dtype=jnp.int32)
    src = jnp.concatenate([edge_index[0].astype(jnp.int32), loop])
    dst = jnp.concatenate([edge_index[1].astype(jnp.int32), loop])
    ne = src.shape[0]
    eidx = jnp.pad(src * PACK + dst, (0, ETOT - ne),
                   constant_values=PAD_DST).reshape(NSUB, CPT, CHUNK)

    xpad = jnp.concatenate([x, jnp.zeros((NPAD - N, D), jnp.float32)], axis=0)
    hh0, es0, ed0, mx0 = _run_layer_tc(_layer0_tc, xpad, W0, a_src0, a_dst0)
    acc0 = _segment_sc(hh0, es0.reshape(NPAD), ed0.reshape(NPAD),
                       mx0.reshape(128), eidx)

    hh1, es1, ed1, mx1 = _run_layer_tc(_layer1_tc, acc0, W1, a_src1, a_dst1,
                                       extra_b=b0)
    acc1 = _segment_sc(hh1, es1.reshape(NPAD), ed1.reshape(NPAD),
                       mx1.reshape(128), eidx)

    node0 = jnp.searchsorted(batch, jnp.arange(BATCH_SIZE, dtype=batch.dtype)
                             ).astype(jnp.int32)
    h64 = pl.pallas_call(
        _head_gather_tc,
        grid_spec=pltpu.PrefetchScalarGridSpec(
            num_scalar_prefetch=1,
            grid=(BATCH_SIZE,),
            in_specs=[
                pl.BlockSpec((2, 1, 1, DEH), lambda j, n0: (0, n0[j], 0, 0)),
                pl.BlockSpec((1, 128), lambda j, n0: (0, 0)),
            ],
            out_specs=pl.BlockSpec((1, 1, 128), lambda j, n0: (j, 0, 0)),
        ),
        out_shape=jax.ShapeDtypeStruct((BATCH_SIZE, 1, 128), jnp.float32),
    )(node0, acc1.reshape(2, ACCR, 1, DEH), b1.reshape(1, 128))
    h64 = h64.reshape(BATCH_SIZE, 128)

    logits = pl.pallas_call(
        _head_mm_tc,
        in_specs=[
            pl.BlockSpec((BATCH_SIZE, 128), lambda: (0, 0)),
            pl.BlockSpec((128, NUM_CLASSES * MAX_SEQ_LEN), lambda: (0, 0)),
            pl.BlockSpec((1, NUM_CLASSES * MAX_SEQ_LEN), lambda: (0, 0)),
        ],
        out_specs=pl.BlockSpec((BATCH_SIZE, NUM_CLASSES * MAX_SEQ_LEN),
                               lambda: (0, 0)),
        out_shape=jax.ShapeDtypeStruct(
            (BATCH_SIZE, NUM_CLASSES * MAX_SEQ_LEN), jnp.float32),
    )(h64, Wp, bp.reshape(1, NUM_CLASSES * MAX_SEQ_LEN))

    return logits.reshape(MAX_SEQ_LEN * BATCH_SIZE, NUM_CLASSES)


# speedup vs baseline: 1.6912x; 1.6912x over previous
import functools

import jax
import jax.numpy as jnp
from jax import lax
from jax.experimental import pallas as pl
from jax.experimental.pallas import tpu as pltpu
from jax.experimental.pallas import tpu_sc as plsc

N = 10000
D = 128
NUM_CLASSES = 32
MAX_SEQ_LEN = 16
BATCH_SIZE = 64

BLK = 1024
GRID = 10
NPAD = BLK * GRID
DEH = 80
SCOL = 48
ACCR = 10112
PAD_DST = N

NSUB = 16
CHUNK = 128
CPT = 162
EPT = CPT * CHUNK
ETOT = NSUB * EPT
RPT = ACCR // NSUB
PACK = 16384


def _layer0_tc(x_ref, w_ref, as_ref, ad_ref, hh_ref, es_ref, ed_ref, mx_ref):
    i = pl.program_id(0)
    xb = x_ref[...]
    h = jnp.dot(xb, w_ref[...], preferred_element_type=jnp.float32)
    _finish_layer_tc(i, h, as_ref, ad_ref, hh_ref, es_ref, ed_ref, mx_ref)


def _finish_layer_tc(i, h, as_ref, ad_ref, hh_ref, es_ref, ed_ref, mx_ref):
    lane = jax.lax.broadcasted_iota(jnp.int32, (BLK, 32), 1)
    extra = jnp.where(lane == 0, 1.0, 0.0).astype(jnp.float32)
    hh_ref[0] = h[:, :DEH]
    hh_ref[1] = jnp.concatenate([h[:, DEH:128], extra], axis=1)
    es = jnp.sum(h * as_ref[...], axis=1, keepdims=True)
    ed = jnp.sum(h * ad_ref[...], axis=1, keepdims=True)
    es_ref[...] = es
    ed_ref[...] = ed
    m = jnp.broadcast_to(jnp.max(es), (1, 128))

    @pl.when(i == 0)
    def _():
        mx_ref[...] = m

    @pl.when(i > 0)
    def _():
        mx_ref[...] = jnp.maximum(mx_ref[...], m)


def _layer1_tc(acc_ref, b0_ref, w_ref, as_ref, ad_ref,
               hh_ref, es_ref, ed_ref, mx_ref):
    i = pl.program_id(0)
    al = acc_ref[0]
    ar = acc_ref[1]
    s = ar[:, SCOL:SCOL + 1]
    x1 = jnp.concatenate([al, ar[:, :SCOL]], axis=1) / s + b0_ref[...]
    x1 = jnp.maximum(x1, 0.0)
    row = jax.lax.broadcasted_iota(jnp.int32, (BLK, 1), 0) + i * BLK
    x1 = jnp.where(row < N, x1, 0.0)
    h = jnp.dot(x1, w_ref[...], preferred_element_type=jnp.float32)
    _finish_layer_tc(i, h, as_ref, ad_ref, hh_ref, es_ref, ed_ref, mx_ref)


def _head_gather_tc(n0_ref, acc_ref, b1_ref, out_ref):
    al = acc_ref[0, 0]
    ar = acc_ref[1, 0]
    s = ar[:, SCOL:SCOL + 1]
    out_ref[0] = jnp.concatenate([al, ar[:, :SCOL]], axis=1) / s + b1_ref[...]


def _head_mm_tc(h_ref, wp_ref, bp_ref, out_ref):
    out_ref[...] = (jnp.dot(h_ref[...], wp_ref[...],
                            preferred_element_type=jnp.float32) + bp_ref[...])


def _run_layer_tc(kern, first_arg, w, a_s, a_d, extra_b=None):
    ins = []
    in_specs = []
    if extra_b is None:
        in_specs.append(pl.BlockSpec((BLK, 128), lambda i: (i, 0)))
        ins.append(first_arg)
    else:
        in_specs.append(pl.BlockSpec((2, BLK, DEH), lambda i: (0, i, 0)))
        ins.append(first_arg)
        in_specs.append(pl.BlockSpec((1, 128), lambda i: (0, 0)))
        ins.append(extra_b.reshape(1, 128))
    in_specs += [
        pl.BlockSpec((128, 128), lambda i: (0, 0)),
        pl.BlockSpec((1, 128), lambda i: (0, 0)),
        pl.BlockSpec((1, 128), lambda i: (0, 0)),
    ]
    ins += [w, a_s.reshape(1, 128), a_d.reshape(1, 128)]
    return pl.pallas_call(
        kern,
        grid=(GRID,),
        in_specs=in_specs,
        out_specs=[
            pl.BlockSpec((2, BLK, DEH), lambda i: (0, i, 0)),
            pl.BlockSpec((BLK, 1), lambda i: (i, 0)),
            pl.BlockSpec((BLK, 1), lambda i: (i, 0)),
            pl.BlockSpec((1, 128), lambda i: (0, 0)),
        ],
        out_shape=[
            jax.ShapeDtypeStruct((2, NPAD, DEH), jnp.float32),
            jax.ShapeDtypeStruct((NPAD, 1), jnp.float32),
            jax.ShapeDtypeStruct((NPAD, 1), jnp.float32),
            jax.ShapeDtypeStruct((1, 128), jnp.float32),
        ],
    )(*ins)


def _make_sc_segment():
    mesh = plsc.VectorSubcoreMesh(core_axis_name="c", subcore_axis_name="s")

    @functools.partial(
        pl.kernel,
        mesh=mesh,
        out_type=jax.ShapeDtypeStruct((2, ACCR, DEH), jnp.float32),
        scratch_types=[
            pltpu.VMEM((NPAD,), jnp.float32),
            pltpu.VMEM((NPAD,), jnp.float32),
            pltpu.VMEM((128,), jnp.float32),
            pltpu.VMEM((CPT, CHUNK), jnp.int32),
            pltpu.VMEM((CHUNK,), jnp.int32),
            pltpu.VMEM((CHUNK,), jnp.int32),
            pltpu.VMEM((CHUNK,), jnp.float32),
            pltpu.VMEM((CHUNK, DEH), jnp.float32),
            pltpu.VMEM((CHUNK,), jnp.int32),
            pltpu.VMEM((CHUNK,), jnp.int32),
            pltpu.VMEM((CHUNK,), jnp.float32),
            pltpu.VMEM((CHUNK, DEH), jnp.float32),
            pltpu.VMEM_SHARED((ACCR, DEH), jnp.float32),
            pltpu.SemaphoreType.DMA,
            pltpu.SemaphoreType.DMA,
            pltpu.SemaphoreType.DMA,
            pltpu.SemaphoreType.DMA,
        ],
        compiler_params=pltpu.CompilerParams(needs_layout_passes=False,
                                             use_tc_tiling_on_sc=False),
    )
    def seg(hh_hbm, es_hbm, ed_hbm, mx_hbm, eidx_hbm, out_hbm,
            es_v, ed_v, mx_v, eidx_v,
            s_a, d_a, w_a, r_a, s_b, d_b, w_b, r_b,
            acc_sh, gsem_a, gsem_b, asem_a, asem_b):
        c = lax.axis_index("c")
        s = lax.axis_index("s")

        pltpu.sync_copy(es_hbm, es_v)
        pltpu.sync_copy(ed_hbm, ed_v)
        pltpu.sync_copy(mx_hbm, mx_v)
        pltpu.sync_copy(eidx_hbm.at[s], eidx_v)
        m16 = mx_v[pl.ds(0, 16)]

        def _zrow(r, carry):
            for j in range(DEH // 16):
                r_a[r, pl.ds(16 * j, 16)] = jnp.zeros((16,), jnp.float32)
                r_b[r, pl.ds(16 * j, 16)] = jnp.zeros((16,), jnp.float32)
            return carry
        lax.fori_loop(0, CHUNK, _zrow, 0)
        for v in range(CHUNK // 16):
            d_a[pl.ds(16 * v, 16)] = jnp.zeros((16,), jnp.int32)
            d_b[pl.ds(16 * v, 16)] = jnp.zeros((16,), jnp.int32)
        for off, sz in ((0, 128), (128, 128), (256, 128), (384, 128),
                        (512, 120)):
            pltpu.sync_copy(r_a.at[pl.ds(0, sz)],
                            acc_sh.at[pl.ds(s * RPT + off, sz)])
        plsc.subcore_barrier()

        pltpu.async_copy(r_a, acc_sh.at[d_a], asem_a, add=True)
        pltpu.async_copy(r_b, acc_sh.at[d_b], asem_b, add=True)

        def _weights(ci, s_x, d_x, w_x):
            def _wvec(v, c2):
                pk = eidx_v[ci, pl.ds(16 * v, 16)]
                si = jax.lax.shift_right_logical(pk, 14)
                di = jax.lax.bitwise_and(pk, PACK - 1)
                s_x[pl.ds(16 * v, 16)] = si
                d_x[pl.ds(16 * v, 16)] = di
                es_g = plsc.load_gather(es_v, [si])
                ed_g = plsc.load_gather(ed_v, [di])
                e = es_g + ed_g
                e = jnp.where(e > 0.0, e, 0.2 * e)
                w_x[pl.ds(16 * v, 16)] = jnp.exp(
                    e - jnp.maximum(ed_g + m16, 0.0))
                return c2
            lax.fori_loop(0, CHUNK // 16, _wvec, 0)

        def _scale(r_x, w_x):
            def _svec(v, c2):
                for l in range(16):
                    r = 16 * v + l
                    ws = plsc.load_gather(w_x, [jnp.full((16,), r, jnp.int32)])
                    ts = [r_x[r, pl.ds(16 * j, 16)] for j in range(DEH // 16)]
                    for j in range(DEH // 16):
                        r_x[r, pl.ds(16 * j, 16)] = ts[j] * ws
                return c2
            lax.fori_loop(0, CHUNK // 16, _svec, 0)

        def _pair(ci2, carry):
            pltpu.make_async_copy(r_a, acc_sh.at[d_a], asem_a).wait()
            _weights(2 * ci2, s_a, d_a, w_a)
            g_a = pltpu.async_copy(hh_hbm.at[c].at[s_a], r_a, gsem_a)

            pltpu.make_async_copy(r_b, acc_sh.at[d_b], asem_b).wait()
            _weights(2 * ci2 + 1, s_b, d_b, w_b)
            g_b = pltpu.async_copy(hh_hbm.at[c].at[s_b], r_b, gsem_b)

            g_a.wait()
            _scale(r_a, w_a)
            pltpu.async_copy(r_a, acc_sh.at[d_a], asem_a, add=True)

            g_b.wait()
            _scale(r_b, w_b)
            pltpu.async_copy(r_b, acc_sh.at[d_b], asem_b, add=True)
            return carry
        lax.fori_loop(0, CPT // 2, _pair, 0)

        pltpu.make_async_copy(r_a, acc_sh.at[d_a], asem_a).wait()
        pltpu.make_async_copy(r_b, acc_sh.at[d_b], asem_b).wait()
        plsc.subcore_barrier()
        pltpu.sync_copy(acc_sh.at[pl.ds(s * RPT, RPT)],
                        out_hbm.at[c, pl.ds(s * RPT, RPT)])

    return seg


_sc_segment = _make_sc_segment()


def _segment_sc(hh, esf, edf, mx, eidx):
    return _sc_segment(hh, esf, edf, mx, eidx)


def kernel(x, edge_index, batch, W0, a_src0, a_dst0, b0,
           W1, a_src1, a_dst1, b1, Wp, bp):
    loop = jnp.arange(N, dtype=jnp.int32)
    src = jnp.concatenate([edge_index[0].astype(jnp.int32), loop])
    dst = jnp.concatenate([edge_index[1].astype(jnp.int32), loop])
    ne = src.shape[0]
    eidx = jnp.pad(src * PACK + dst, (0, ETOT - ne),
                   constant_values=PAD_DST).reshape(NSUB, CPT, CHUNK)

    xpad = jnp.concatenate([x, jnp.zeros((NPAD - N, D), jnp.float32)], axis=0)
    hh0, es0, ed0, mx0 = _run_layer_tc(_layer0_tc, xpad, W0, a_src0, a_dst0)
    acc0 = _segment_sc(hh0, es0.reshape(NPAD), ed0.reshape(NPAD),
                       mx0.reshape(128), eidx)

    hh1, es1, ed1, mx1 = _run_layer_tc(_layer1_tc, acc0, W1, a_src1, a_dst1,
                                       extra_b=b0)
    acc1 = _segment_sc(hh1, es1.reshape(NPAD), ed1.reshape(NPAD),
                       mx1.reshape(128), eidx)

    node0 = jnp.searchsorted(batch, jnp.arange(BATCH_SIZE, dtype=batch.dtype)
                             ).astype(jnp.int32)
    h64 = pl.pallas_call(
        _head_gather_tc,
        grid_spec=pltpu.PrefetchScalarGridSpec(
            num_scalar_prefetch=1,
            grid=(BATCH_SIZE,),
            in_specs=[
                pl.BlockSpec((2, 1, 1, DEH), lambda j, n0: (0, n0[j], 0, 0)),
                pl.BlockSpec((1, 128), lambda j, n0: (0, 0)),
            ],
            out_specs=pl.BlockSpec((1, 1, 128), lambda j, n0: (j, 0, 0)),
        ),
        out_shape=jax.ShapeDtypeStruct((BATCH_SIZE, 1, 128), jnp.float32),
    )(node0, acc1.reshape(2, ACCR, 1, DEH), b1.reshape(1, 128))
    h64 = h64.reshape(BATCH_SIZE, 128)

    logits = pl.pallas_call(
        _head_mm_tc,
        in_specs=[
            pl.BlockSpec((BATCH_SIZE, 128), lambda: (0, 0)),
            pl.BlockSpec((128, NUM_CLASSES * MAX_SEQ_LEN), lambda: (0, 0)),
            pl.BlockSpec((1, NUM_CLASSES * MAX_SEQ_LEN), lambda: (0, 0)),
        ],
        out_specs=pl.BlockSpec((BATCH_SIZE, NUM_CLASSES * MAX_SEQ_LEN),
                               lambda: (0, 0)),
        out_shape=jax.ShapeDtypeStruct(
            (BATCH_SIZE, NUM_CLASSES * MAX_SEQ_LEN), jnp.float32),
    )(h64, Wp, bp.reshape(1, NUM_CLASSES * MAX_SEQ_LEN))

    return logits.reshape(MAX_SEQ_LEN * BATCH_SIZE, NUM_CLASSES)


# speedup vs baseline: 1.9620x; 1.1601x over previous
import functools

import jax
import jax.numpy as jnp
from jax import lax
from jax.experimental import pallas as pl
from jax.experimental.pallas import tpu as pltpu
from jax.experimental.pallas import tpu_sc as plsc

N = 10000
D = 128
NUM_CLASSES = 32
MAX_SEQ_LEN = 16
BATCH_SIZE = 64

BLK = 1024
GRID = 10
NPAD = BLK * GRID
DEH = 80
SCOL = 48
ACCR = 10112
PAD_DST = N

NSUB = 16
CHUNK = 128
CPT = 162
EPT = CPT * CHUNK
ETOT = NSUB * EPT
RPT = ACCR // NSUB
PACK = 16384


def _layer0_tc(x_ref, w_ref, as_ref, ad_ref, hh_ref, es_ref, ed_ref, mx_ref):
    i = pl.program_id(0)
    xb = x_ref[...]
    h = jnp.dot(xb, w_ref[...], preferred_element_type=jnp.float32)
    _finish_layer_tc(i, h, as_ref, ad_ref, hh_ref, es_ref, ed_ref, mx_ref)


def _finish_layer_tc(i, h, as_ref, ad_ref, hh_ref, es_ref, ed_ref, mx_ref):
    lane = jax.lax.broadcasted_iota(jnp.int32, (BLK, 32), 1)
    extra = jnp.where(lane == 0, 1.0, 0.0).astype(jnp.float32)
    hh_ref[0] = h[:, :DEH]
    hh_ref[1] = jnp.concatenate([h[:, DEH:128], extra], axis=1)
    es = jnp.sum(h * as_ref[...], axis=1, keepdims=True)
    ed = jnp.sum(h * ad_ref[...], axis=1, keepdims=True)
    es_ref[...] = es
    ed_ref[...] = ed
    m = jnp.broadcast_to(jnp.max(es), (1, 128))

    @pl.when(i == 0)
    def _():
        mx_ref[...] = m

    @pl.when(i > 0)
    def _():
        mx_ref[...] = jnp.maximum(mx_ref[...], m)


def _layer1_tc(acc_ref, b0_ref, w_ref, as_ref, ad_ref,
               hh_ref, es_ref, ed_ref, mx_ref):
    i = pl.program_id(0)
    al = acc_ref[0]
    ar = acc_ref[1]
    s = ar[:, SCOL:SCOL + 1]
    x1 = jnp.concatenate([al, ar[:, :SCOL]], axis=1) / s + b0_ref[...]
    x1 = jnp.maximum(x1, 0.0)
    row = jax.lax.broadcasted_iota(jnp.int32, (BLK, 1), 0) + i * BLK
    x1 = jnp.where(row < N, x1, 0.0)
    h = jnp.dot(x1, w_ref[...], preferred_element_type=jnp.float32)
    _finish_layer_tc(i, h, as_ref, ad_ref, hh_ref, es_ref, ed_ref, mx_ref)


def _head_gather_tc(n0_ref, acc_ref, b1_ref, out_ref):
    al = acc_ref[0, 0]
    ar = acc_ref[1, 0]
    s = ar[:, SCOL:SCOL + 1]
    out_ref[0] = jnp.concatenate([al, ar[:, :SCOL]], axis=1) / s + b1_ref[...]


def _head_mm_tc(h_ref, wp_ref, bp_ref, out_ref):
    out_ref[...] = (jnp.dot(h_ref[...], wp_ref[...],
                            preferred_element_type=jnp.float32) + bp_ref[...])


def _run_layer_tc(kern, first_arg, w, a_s, a_d, extra_b=None):
    ins = []
    in_specs = []
    if extra_b is None:
        in_specs.append(pl.BlockSpec((BLK, 128), lambda i: (i, 0)))
        ins.append(first_arg)
    else:
        in_specs.append(pl.BlockSpec((2, BLK, DEH), lambda i: (0, i, 0)))
        ins.append(first_arg)
        in_specs.append(pl.BlockSpec((1, 128), lambda i: (0, 0)))
        ins.append(extra_b.reshape(1, 128))
    in_specs += [
        pl.BlockSpec((128, 128), lambda i: (0, 0)),
        pl.BlockSpec((1, 128), lambda i: (0, 0)),
        pl.BlockSpec((1, 128), lambda i: (0, 0)),
    ]
    ins += [w, a_s.reshape(1, 128), a_d.reshape(1, 128)]
    return pl.pallas_call(
        kern,
        grid=(GRID,),
        in_specs=in_specs,
        out_specs=[
            pl.BlockSpec((2, BLK, DEH), lambda i: (0, i, 0)),
            pl.BlockSpec((BLK, 1), lambda i: (i, 0)),
            pl.BlockSpec((BLK, 1), lambda i: (i, 0)),
            pl.BlockSpec((1, 128), lambda i: (0, 0)),
        ],
        out_shape=[
            jax.ShapeDtypeStruct((2, NPAD, DEH), jnp.float32),
            jax.ShapeDtypeStruct((NPAD, 1), jnp.float32),
            jax.ShapeDtypeStruct((NPAD, 1), jnp.float32),
            jax.ShapeDtypeStruct((1, 128), jnp.float32),
        ],
    )(*ins)


def _make_sc_segment():
    mesh = plsc.VectorSubcoreMesh(core_axis_name="c", subcore_axis_name="s")

    @functools.partial(
        pl.kernel,
        mesh=mesh,
        out_type=jax.ShapeDtypeStruct((2, ACCR, DEH), jnp.float32),
        scratch_types=[
            pltpu.VMEM((NPAD,), jnp.float32),
            pltpu.VMEM((NPAD,), jnp.float32),
            pltpu.VMEM((128,), jnp.float32),
            pltpu.VMEM((CPT, CHUNK), jnp.int32),
            pltpu.VMEM((CHUNK,), jnp.int32),
            pltpu.VMEM((CHUNK,), jnp.int32),
            pltpu.VMEM((CHUNK,), jnp.float32),
            pltpu.VMEM((CHUNK, DEH), jnp.float32),
            pltpu.VMEM((CHUNK,), jnp.int32),
            pltpu.VMEM((CHUNK,), jnp.int32),
            pltpu.VMEM((CHUNK,), jnp.float32),
            pltpu.VMEM((CHUNK, DEH), jnp.float32),
            pltpu.VMEM_SHARED((ACCR, DEH), jnp.float32),
            pltpu.SemaphoreType.DMA,
            pltpu.SemaphoreType.DMA,
            pltpu.SemaphoreType.DMA,
            pltpu.SemaphoreType.DMA,
        ],
        compiler_params=pltpu.CompilerParams(needs_layout_passes=False,
                                             use_tc_tiling_on_sc=False),
    )
    def seg(hh_hbm, es_hbm, ed_hbm, mx_hbm, eidx_hbm, out_hbm,
            es_v, ed_v, mx_v, eidx_v,
            s_a, d_a, w_a, r_a, s_b, d_b, w_b, r_b,
            acc_sh, gsem_a, gsem_b, asem_a, asem_b):
        c = lax.axis_index("c")
        s = lax.axis_index("s")

        pltpu.sync_copy(es_hbm, es_v)
        pltpu.sync_copy(ed_hbm, ed_v)
        pltpu.sync_copy(mx_hbm, mx_v)
        pltpu.sync_copy(eidx_hbm.at[s], eidx_v)
        m16 = mx_v[pl.ds(0, 16)]

        def _zrow(r, carry):
            for j in range(DEH // 16):
                r_a[r, pl.ds(16 * j, 16)] = jnp.zeros((16,), jnp.float32)
                r_b[r, pl.ds(16 * j, 16)] = jnp.zeros((16,), jnp.float32)
            return carry
        lax.fori_loop(0, CHUNK, _zrow, 0)
        for v in range(CHUNK // 16):
            d_a[pl.ds(16 * v, 16)] = jnp.zeros((16,), jnp.int32)
            d_b[pl.ds(16 * v, 16)] = jnp.zeros((16,), jnp.int32)
        for off, sz in ((0, 128), (128, 128), (256, 128), (384, 128),
                        (512, 120)):
            pltpu.sync_copy(r_a.at[pl.ds(0, sz)],
                            acc_sh.at[pl.ds(s * RPT + off, sz)])
        plsc.subcore_barrier()

        pltpu.async_copy(r_a, acc_sh.at[d_a], asem_a, add=True)
        pltpu.async_copy(r_b, acc_sh.at[d_b], asem_b, add=True)

        def _weights(ci, s_x, d_x, w_x):
            def _wvec(v, c2):
                pk = eidx_v[ci, pl.ds(16 * v, 16)]
                si = jax.lax.shift_right_logical(pk, 14)
                di = jax.lax.bitwise_and(pk, PACK - 1)
                s_x[pl.ds(16 * v, 16)] = si
                d_x[pl.ds(16 * v, 16)] = di
                es_g = plsc.load_gather(es_v, [si])
                ed_g = plsc.load_gather(ed_v, [di])
                e = es_g + ed_g
                e = jnp.where(e > 0.0, e, 0.2 * e)
                w_x[pl.ds(16 * v, 16)] = jnp.exp(
                    e - jnp.maximum(ed_g + m16, 0.0))
                return c2
            lax.fori_loop(0, CHUNK // 16, _wvec, 0)

        def _scale(r_x, w_x):
            def _svec(v, c2):
                wv = w_x[pl.ds(16 * v, 16)]
                for l in range(16):
                    r = 16 * v + l
                    ws = lax.gather(
                        wv, jnp.full((16, 1), l, jnp.int32),
                        lax.GatherDimensionNumbers(
                            offset_dims=(), collapsed_slice_dims=(0,),
                            start_index_map=(0,)),
                        slice_sizes=(1,),
                        mode=lax.GatherScatterMode.PROMISE_IN_BOUNDS)
                    ts = [r_x[r, pl.ds(16 * j, 16)] for j in range(DEH // 16)]
                    for j in range(DEH // 16):
                        r_x[r, pl.ds(16 * j, 16)] = ts[j] * ws
                return c2
            lax.fori_loop(0, CHUNK // 16, _svec, 0)

        def _pair(ci2, carry):
            pltpu.make_async_copy(r_a, acc_sh.at[d_a], asem_a).wait()
            _weights(2 * ci2, s_a, d_a, w_a)
            g_a = pltpu.async_copy(hh_hbm.at[c].at[s_a], r_a, gsem_a)

            pltpu.make_async_copy(r_b, acc_sh.at[d_b], asem_b).wait()
            _weights(2 * ci2 + 1, s_b, d_b, w_b)
            g_b = pltpu.async_copy(hh_hbm.at[c].at[s_b], r_b, gsem_b)

            g_a.wait()
            _scale(r_a, w_a)
            pltpu.async_copy(r_a, acc_sh.at[d_a], asem_a, add=True)

            g_b.wait()
            _scale(r_b, w_b)
            pltpu.async_copy(r_b, acc_sh.at[d_b], asem_b, add=True)
            return carry
        lax.fori_loop(0, CPT // 2, _pair, 0)

        pltpu.make_async_copy(r_a, acc_sh.at[d_a], asem_a).wait()
        pltpu.make_async_copy(r_b, acc_sh.at[d_b], asem_b).wait()
        plsc.subcore_barrier()
        pltpu.sync_copy(acc_sh.at[pl.ds(s * RPT, RPT)],
                        out_hbm.at[c, pl.ds(s * RPT, RPT)])

    return seg


_sc_segment = _make_sc_segment()


def _segment_sc(hh, esf, edf, mx, eidx):
    return _sc_segment(hh, esf, edf, mx, eidx)


def kernel(x, edge_index, batch, W0, a_src0, a_dst0, b0,
           W1, a_src1, a_dst1, b1, Wp, bp):
    loop = jnp.arange(N, dtype=jnp.int32)
    src = jnp.concatenate([edge_index[0].astype(jnp.int32), loop])
    dst = jnp.concatenate([edge_index[1].astype(jnp.int32), loop])
    ne = src.shape[0]
    eidx = jnp.pad(src * PACK + dst, (0, ETOT - ne),
                   constant_values=PAD_DST).reshape(NSUB, CPT, CHUNK)

    xpad = jnp.concatenate([x, jnp.zeros((NPAD - N, D), jnp.float32)], axis=0)
    hh0, es0, ed0, mx0 = _run_layer_tc(_layer0_tc, xpad, W0, a_src0, a_dst0)
    acc0 = _segment_sc(hh0, es0.reshape(NPAD), ed0.reshape(NPAD),
                       mx0.reshape(128), eidx)

    hh1, es1, ed1, mx1 = _run_layer_tc(_layer1_tc, acc0, W1, a_src1, a_dst1,
                                       extra_b=b0)
    acc1 = _segment_sc(hh1, es1.reshape(NPAD), ed1.reshape(NPAD),
                       mx1.reshape(128), eidx)

    node0 = jnp.searchsorted(batch, jnp.arange(BATCH_SIZE, dtype=batch.dtype)
                             ).astype(jnp.int32)
    h64 = pl.pallas_call(
        _head_gather_tc,
        grid_spec=pltpu.PrefetchScalarGridSpec(
            num_scalar_prefetch=1,
            grid=(BATCH_SIZE,),
            in_specs=[
                pl.BlockSpec((2, 1, 1, DEH), lambda j, n0: (0, n0[j], 0, 0)),
                pl.BlockSpec((1, 128), lambda j, n0: (0, 0)),
            ],
            out_specs=pl.BlockSpec((1, 1, 128), lambda j, n0: (j, 0, 0)),
        ),
        out_shape=jax.ShapeDtypeStruct((BATCH_SIZE, 1, 128), jnp.float32),
    )(node0, acc1.reshape(2, ACCR, 1, DEH), b1.reshape(1, 128))
    h64 = h64.reshape(BATCH_SIZE, 128)

    logits = pl.pallas_call(
        _head_mm_tc,
        in_specs=[
            pl.BlockSpec((BATCH_SIZE, 128), lambda: (0, 0)),
            pl.BlockSpec((128, NUM_CLASSES * MAX_SEQ_LEN), lambda: (0, 0)),
            pl.BlockSpec((1, NUM_CLASSES * MAX_SEQ_LEN), lambda: (0, 0)),
        ],
        out_specs=pl.BlockSpec((BATCH_SIZE, NUM_CLASSES * MAX_SEQ_LEN),
                               lambda: (0, 0)),
        out_shape=jax.ShapeDtypeStruct(
            (BATCH_SIZE, NUM_CLASSES * MAX_SEQ_LEN), jnp.float32),
    )(h64, Wp, bp.reshape(1, NUM_CLASSES * MAX_SEQ_LEN))

    return logits.reshape(MAX_SEQ_LEN * BATCH_SIZE, NUM_CLASSES)


# speedup vs baseline: 1.9892x; 1.0139x over previous
import functools

import jax
import jax.numpy as jnp
from jax import lax
from jax.experimental import pallas as pl
from jax.experimental.pallas import tpu as pltpu
from jax.experimental.pallas import tpu_sc as plsc

N = 10000
D = 128
NUM_CLASSES = 32
MAX_SEQ_LEN = 16
BATCH_SIZE = 64

BLK = 1024
GRID = 10
NPAD = BLK * GRID
DEH = 80
SCOL = 48
ACCR = 10112
PAD_DST = N

NSUB = 16
CHUNK = 128
CPT = 162
EPT = CPT * CHUNK
ETOT = NSUB * EPT
RPT = ACCR // NSUB
PACK = 16384


def _layer0_tc(x_ref, w_ref, as_ref, ad_ref, hh_ref, es_ref, ed_ref, mx_ref):
    i = pl.program_id(0)
    xb = x_ref[...]
    h = jnp.dot(xb, w_ref[...], preferred_element_type=jnp.float32)
    _finish_layer_tc(i, h, as_ref, ad_ref, hh_ref, es_ref, ed_ref, mx_ref)


def _finish_layer_tc(i, h, as_ref, ad_ref, hh_ref, es_ref, ed_ref, mx_ref):
    lane = jax.lax.broadcasted_iota(jnp.int32, (BLK, 32), 1)
    extra = jnp.where(lane == 0, 1.0, 0.0).astype(jnp.float32)
    hh_ref[0] = h[:, :DEH]
    hh_ref[1] = jnp.concatenate([h[:, DEH:128], extra], axis=1)
    es = jnp.sum(h * as_ref[...], axis=1, keepdims=True)
    ed = jnp.sum(h * ad_ref[...], axis=1, keepdims=True)
    es_ref[...] = es
    ed_ref[...] = ed
    row = jax.lax.broadcasted_iota(jnp.int32, (BLK, 1), 0) + i * BLK
    m = jnp.broadcast_to(jnp.max(jnp.where(row < N, es, -3e38)), (1, 128))

    @pl.when(i == 0)
    def _():
        mx_ref[...] = m

    @pl.when(i > 0)
    def _():
        mx_ref[...] = jnp.maximum(mx_ref[...], m)


def _layer1_tc(acc_ref, b0_ref, w_ref, as_ref, ad_ref,
               hh_ref, es_ref, ed_ref, mx_ref):
    i = pl.program_id(0)
    al = acc_ref[0]
    ar = acc_ref[1]
    s = ar[:, SCOL:SCOL + 1]
    x1 = jnp.concatenate([al, ar[:, :SCOL]], axis=1) / s + b0_ref[...]
    x1 = jnp.maximum(x1, 0.0)
    row = jax.lax.broadcasted_iota(jnp.int32, (BLK, 1), 0) + i * BLK
    x1 = jnp.where(row < N, x1, 0.0)
    h = jnp.dot(x1, w_ref[...], preferred_element_type=jnp.float32)
    _finish_layer_tc(i, h, as_ref, ad_ref, hh_ref, es_ref, ed_ref, mx_ref)


def _head_gather_tc(n0_ref, acc_ref, b1_ref, out_ref):
    al = acc_ref[0, 0]
    ar = acc_ref[1, 0]
    s = ar[:, SCOL:SCOL + 1]
    out_ref[0] = jnp.concatenate([al, ar[:, :SCOL]], axis=1) / s + b1_ref[...]


def _head_mm_tc(h_ref, wp_ref, bp_ref, out_ref):
    out_ref[...] = (jnp.dot(h_ref[...], wp_ref[...],
                            preferred_element_type=jnp.float32) + bp_ref[...])


def _run_layer_tc(kern, first_arg, w, a_s, a_d, extra_b=None):
    ins = []
    in_specs = []
    if extra_b is None:
        in_specs.append(pl.BlockSpec((BLK, 128), lambda i: (i, 0)))
        ins.append(first_arg)
    else:
        in_specs.append(pl.BlockSpec((2, BLK, DEH), lambda i: (0, i, 0)))
        ins.append(first_arg)
        in_specs.append(pl.BlockSpec((1, 128), lambda i: (0, 0)))
        ins.append(extra_b.reshape(1, 128))
    in_specs += [
        pl.BlockSpec((128, 128), lambda i: (0, 0)),
        pl.BlockSpec((1, 128), lambda i: (0, 0)),
        pl.BlockSpec((1, 128), lambda i: (0, 0)),
    ]
    ins += [w, a_s.reshape(1, 128), a_d.reshape(1, 128)]
    return pl.pallas_call(
        kern,
        grid=(GRID,),
        in_specs=in_specs,
        out_specs=[
            pl.BlockSpec((2, BLK, DEH), lambda i: (0, i, 0)),
            pl.BlockSpec((BLK, 1), lambda i: (i, 0)),
            pl.BlockSpec((BLK, 1), lambda i: (i, 0)),
            pl.BlockSpec((1, 128), lambda i: (0, 0)),
        ],
        out_shape=[
            jax.ShapeDtypeStruct((2, NPAD, DEH), jnp.float32),
            jax.ShapeDtypeStruct((NPAD, 1), jnp.float32),
            jax.ShapeDtypeStruct((NPAD, 1), jnp.float32),
            jax.ShapeDtypeStruct((1, 128), jnp.float32),
        ],
    )(*ins)


def _make_sc_segment():
    mesh = plsc.VectorSubcoreMesh(core_axis_name="c", subcore_axis_name="s")

    @functools.partial(
        pl.kernel,
        mesh=mesh,
        out_type=jax.ShapeDtypeStruct((2, ACCR, DEH), jnp.float32),
        scratch_types=[
            pltpu.VMEM((NPAD,), jnp.float32),
            pltpu.VMEM((NPAD,), jnp.float32),
            pltpu.VMEM((128,), jnp.float32),
            pltpu.VMEM((CPT, CHUNK), jnp.int32),
            pltpu.VMEM((CHUNK,), jnp.int32),
            pltpu.VMEM((CHUNK,), jnp.int32),
            pltpu.VMEM((CHUNK,), jnp.float32),
            pltpu.VMEM((CHUNK, DEH), jnp.float32),
            pltpu.VMEM((CHUNK,), jnp.int32),
            pltpu.VMEM((CHUNK,), jnp.int32),
            pltpu.VMEM((CHUNK,), jnp.float32),
            pltpu.VMEM((CHUNK, DEH), jnp.float32),
            pltpu.VMEM_SHARED((ACCR, DEH), jnp.float32),
            pltpu.SemaphoreType.DMA,
            pltpu.SemaphoreType.DMA,
            pltpu.SemaphoreType.DMA,
            pltpu.SemaphoreType.DMA,
        ],
        compiler_params=pltpu.CompilerParams(needs_layout_passes=False,
                                             use_tc_tiling_on_sc=False),
    )
    def seg(hh_hbm, es_hbm, ed_hbm, mx_hbm, eidx_hbm, out_hbm,
            es_v, ed_v, mx_v, eidx_v,
            s_a, d_a, w_a, r_a, s_b, d_b, w_b, r_b,
            acc_sh, gsem_a, gsem_b, asem_a, asem_b):
        c = lax.axis_index("c")
        s = lax.axis_index("s")

        pltpu.sync_copy(es_hbm, es_v)
        pltpu.sync_copy(ed_hbm, ed_v)
        pltpu.sync_copy(mx_hbm, mx_v)
        pltpu.sync_copy(eidx_hbm.at[s], eidx_v)
        m16 = mx_v[pl.ds(0, 16)]

        def _zrow(r, carry):
            for j in range(DEH // 16):
                r_a[r, pl.ds(16 * j, 16)] = jnp.zeros((16,), jnp.float32)
                r_b[r, pl.ds(16 * j, 16)] = jnp.zeros((16,), jnp.float32)
            return carry
        lax.fori_loop(0, CHUNK, _zrow, 0)
        for v in range(CHUNK // 16):
            d_a[pl.ds(16 * v, 16)] = jnp.zeros((16,), jnp.int32)
            d_b[pl.ds(16 * v, 16)] = jnp.zeros((16,), jnp.int32)
        for off, sz in ((0, 128), (128, 128), (256, 128), (384, 128),
                        (512, 120)):
            pltpu.sync_copy(r_a.at[pl.ds(0, sz)],
                            acc_sh.at[pl.ds(s * RPT + off, sz)])
        plsc.subcore_barrier()

        pltpu.async_copy(r_a, acc_sh.at[d_a], asem_a, add=True)
        pltpu.async_copy(r_b, acc_sh.at[d_b], asem_b, add=True)

        def _weights(ci, s_x, d_x, w_x):
            def _wvec(v, c2):
                pk = eidx_v[ci, pl.ds(16 * v, 16)]
                si = jax.lax.shift_right_logical(pk, 14)
                di = jax.lax.bitwise_and(pk, PACK - 1)
                s_x[pl.ds(16 * v, 16)] = si
                d_x[pl.ds(16 * v, 16)] = di
                es_g = plsc.load_gather(es_v, [si])
                ed_g = plsc.load_gather(ed_v, [di])
                e = es_g + ed_g
                e = jnp.where(e > 0.0, e, 0.2 * e)
                w_x[pl.ds(16 * v, 16)] = jnp.exp(
                    e - jnp.maximum(ed_g + m16, 0.0))
                return c2
            lax.fori_loop(0, CHUNK // 16, _wvec, 0)

        def _lanebc(wv, l):
            return lax.gather(
                wv, jnp.full((16, 1), l, jnp.int32),
                lax.GatherDimensionNumbers(
                    offset_dims=(), collapsed_slice_dims=(0,),
                    start_index_map=(0,)),
                slice_sizes=(1,),
                mode=lax.GatherScatterMode.PROMISE_IN_BOUNDS)

        def _scale(r_x, w_x):
            def _svec(v, c2):
                wv = w_x[pl.ds(16 * v, 16)]
                for l in range(0, 16, 2):
                    r0 = 16 * v + l
                    r1 = r0 + 1
                    ws0 = _lanebc(wv, l)
                    ws1 = _lanebc(wv, l + 1)
                    t0 = [r_x[r0, pl.ds(16 * j, 16)] for j in range(DEH // 16)]
                    t1 = [r_x[r1, pl.ds(16 * j, 16)] for j in range(DEH // 16)]
                    for j in range(DEH // 16):
                        r_x[r0, pl.ds(16 * j, 16)] = t0[j] * ws0
                    for j in range(DEH // 16):
                        r_x[r1, pl.ds(16 * j, 16)] = t1[j] * ws1
                return c2
            lax.fori_loop(0, CHUNK // 16, _svec, 0)

        def _pair(ci2, carry):
            pltpu.make_async_copy(r_a, acc_sh.at[d_a], asem_a).wait()
            _weights(2 * ci2, s_a, d_a, w_a)
            g_a = pltpu.async_copy(hh_hbm.at[c].at[s_a], r_a, gsem_a)

            pltpu.make_async_copy(r_b, acc_sh.at[d_b], asem_b).wait()
            _weights(2 * ci2 + 1, s_b, d_b, w_b)
            g_b = pltpu.async_copy(hh_hbm.at[c].at[s_b], r_b, gsem_b)

            g_a.wait()
            _scale(r_a, w_a)
            pltpu.async_copy(r_a, acc_sh.at[d_a], asem_a, add=True)

            g_b.wait()
            _scale(r_b, w_b)
            pltpu.async_copy(r_b, acc_sh.at[d_b], asem_b, add=True)
            return carry
        lax.fori_loop(0, CPT // 2, _pair, 0)

        pltpu.make_async_copy(r_a, acc_sh.at[d_a], asem_a).wait()
        pltpu.make_async_copy(r_b, acc_sh.at[d_b], asem_b).wait()
        plsc.subcore_barrier()
        pltpu.sync_copy(acc_sh.at[pl.ds(s * RPT, RPT)],
                        out_hbm.at[c, pl.ds(s * RPT, RPT)])

    return seg


_sc_segment = _make_sc_segment()


def _segment_sc(hh, esf, edf, mx, eidx):
    return _sc_segment(hh, esf, edf, mx, eidx)


def kernel(x, edge_index, batch, W0, a_src0, a_dst0, b0,
           W1, a_src1, a_dst1, b1, Wp, bp):
    loop = jnp.arange(N, dtype=jnp.int32)
    src = jnp.concatenate([edge_index[0].astype(jnp.int32), loop])
    dst = jnp.concatenate([edge_index[1].astype(jnp.int32), loop])
    ne = src.shape[0]
    eidx = jnp.pad(src * PACK + dst, (0, ETOT - ne),
                   constant_values=PAD_DST).reshape(NSUB, CPT, CHUNK)

    hh0, es0, ed0, mx0 = _run_layer_tc(_layer0_tc, x, W0, a_src0, a_dst0)
    acc0 = _segment_sc(hh0, es0.reshape(NPAD), ed0.reshape(NPAD),
                       mx0.reshape(128), eidx)

    hh1, es1, ed1, mx1 = _run_layer_tc(_layer1_tc, acc0, W1, a_src1, a_dst1,
                                       extra_b=b0)
    acc1 = _segment_sc(hh1, es1.reshape(NPAD), ed1.reshape(NPAD),
                       mx1.reshape(128), eidx)

    node0 = jnp.searchsorted(batch, jnp.arange(BATCH_SIZE, dtype=batch.dtype)
                             ).astype(jnp.int32)
    h64 = pl.pallas_call(
        _head_gather_tc,
        grid_spec=pltpu.PrefetchScalarGridSpec(
            num_scalar_prefetch=1,
            grid=(BATCH_SIZE,),
            in_specs=[
                pl.BlockSpec((2, 1, 1, DEH), lambda j, n0: (0, n0[j], 0, 0)),
                pl.BlockSpec((1, 128), lambda j, n0: (0, 0)),
            ],
            out_specs=pl.BlockSpec((1, 1, 128), lambda j, n0: (j, 0, 0)),
        ),
        out_shape=jax.ShapeDtypeStruct((BATCH_SIZE, 1, 128), jnp.float32),
    )(node0, acc1.reshape(2, ACCR, 1, DEH), b1.reshape(1, 128))
    h64 = h64.reshape(BATCH_SIZE, 128)

    logits = pl.pallas_call(
        _head_mm_tc,
        in_specs=[
            pl.BlockSpec((BATCH_SIZE, 128), lambda: (0, 0)),
            pl.BlockSpec((128, NUM_CLASSES * MAX_SEQ_LEN), lambda: (0, 0)),
            pl.BlockSpec((1, NUM_CLASSES * MAX_SEQ_LEN), lambda: (0, 0)),
        ],
        out_specs=pl.BlockSpec((BATCH_SIZE, NUM_CLASSES * MAX_SEQ_LEN),
                               lambda: (0, 0)),
        out_shape=jax.ShapeDtypeStruct(
            (BATCH_SIZE, NUM_CLASSES * MAX_SEQ_LEN), jnp.float32),
    )(h64, Wp, bp.reshape(1, NUM_CLASSES * MAX_SEQ_LEN))

    return logits.reshape(MAX_SEQ_LEN * BATCH_SIZE, NUM_CLASSES)


# speedup vs baseline: 2.1493x; 1.0805x over previous
import functools

import jax
import jax.numpy as jnp
from jax import lax
from jax.experimental import pallas as pl
from jax.experimental.pallas import tpu as pltpu
from jax.experimental.pallas import tpu_sc as plsc

N = 10000
D = 128
NUM_CLASSES = 32
MAX_SEQ_LEN = 16
BATCH_SIZE = 64

BLK = 1024
GRID = 10
NPAD = BLK * GRID
DEH = 80
SCOL = 48
ACCR = 10112
PAD_DST = N

NSUB = 16
CHUNK = 128
CPT = 162
CPTX = CPT + 2
EPT = CPT * CHUNK
ETOT = NSUB * EPT
RPT = ACCR // NSUB
PACK = 16384


def _layer0_tc(x_ref, w_ref, as_ref, ad_ref, hh_ref, es_ref, ed_ref, mx_ref):
    i = pl.program_id(0)
    xb = x_ref[...]
    h = jnp.dot(xb, w_ref[...], preferred_element_type=jnp.float32)
    _finish_layer_tc(i, h, as_ref, ad_ref, hh_ref, es_ref, ed_ref, mx_ref)


def _finish_layer_tc(i, h, as_ref, ad_ref, hh_ref, es_ref, ed_ref, mx_ref):
    lane = jax.lax.broadcasted_iota(jnp.int32, (BLK, 32), 1)
    extra = jnp.where(lane == 0, 1.0, 0.0).astype(jnp.float32)
    hh_ref[0] = h[:, :DEH]
    hh_ref[1] = jnp.concatenate([h[:, DEH:128], extra], axis=1)
    es = jnp.sum(h * as_ref[...], axis=1, keepdims=True)
    ed = jnp.sum(h * ad_ref[...], axis=1, keepdims=True)
    es_ref[...] = es
    ed_ref[...] = ed
    row = jax.lax.broadcasted_iota(jnp.int32, (BLK, 1), 0) + i * BLK
    m = jnp.broadcast_to(jnp.max(jnp.where(row < N, es, -3e38)), (1, 128))

    @pl.when(i == 0)
    def _():
        mx_ref[...] = m

    @pl.when(i > 0)
    def _():
        mx_ref[...] = jnp.maximum(mx_ref[...], m)


def _layer1_tc(acc_ref, b0_ref, w_ref, as_ref, ad_ref,
               hh_ref, es_ref, ed_ref, mx_ref):
    i = pl.program_id(0)
    al = acc_ref[0]
    ar = acc_ref[1]
    s = ar[:, SCOL:SCOL + 1]
    x1 = jnp.concatenate([al, ar[:, :SCOL]], axis=1) / s + b0_ref[...]
    x1 = jnp.maximum(x1, 0.0)
    row = jax.lax.broadcasted_iota(jnp.int32, (BLK, 1), 0) + i * BLK
    x1 = jnp.where(row < N, x1, 0.0)
    h = jnp.dot(x1, w_ref[...], preferred_element_type=jnp.float32)
    _finish_layer_tc(i, h, as_ref, ad_ref, hh_ref, es_ref, ed_ref, mx_ref)


def _head_gather_tc(n0_ref, acc_ref, b1_ref, out_ref):
    al = acc_ref[0, 0]
    ar = acc_ref[1, 0]
    s = ar[:, SCOL:SCOL + 1]
    out_ref[0] = jnp.concatenate([al, ar[:, :SCOL]], axis=1) / s + b1_ref[...]


def _head_mm_tc(h_ref, wp_ref, bp_ref, out_ref):
    out_ref[...] = (jnp.dot(h_ref[...], wp_ref[...],
                            preferred_element_type=jnp.float32) + bp_ref[...])


def _run_layer_tc(kern, first_arg, w, a_s, a_d, extra_b=None):
    ins = []
    in_specs = []
    if extra_b is None:
        in_specs.append(pl.BlockSpec((BLK, 128), lambda i: (i, 0)))
        ins.append(first_arg)
    else:
        in_specs.append(pl.BlockSpec((2, BLK, DEH), lambda i: (0, i, 0)))
        ins.append(first_arg)
        in_specs.append(pl.BlockSpec((1, 128), lambda i: (0, 0)))
        ins.append(extra_b.reshape(1, 128))
    in_specs += [
        pl.BlockSpec((128, 128), lambda i: (0, 0)),
        pl.BlockSpec((1, 128), lambda i: (0, 0)),
        pl.BlockSpec((1, 128), lambda i: (0, 0)),
    ]
    ins += [w, a_s.reshape(1, 128), a_d.reshape(1, 128)]
    return pl.pallas_call(
        kern,
        grid=(GRID,),
        in_specs=in_specs,
        out_specs=[
            pl.BlockSpec((2, BLK, DEH), lambda i: (0, i, 0)),
            pl.BlockSpec((BLK, 1), lambda i: (i, 0)),
            pl.BlockSpec((BLK, 1), lambda i: (i, 0)),
            pl.BlockSpec((1, 128), lambda i: (0, 0)),
        ],
        out_shape=[
            jax.ShapeDtypeStruct((2, NPAD, DEH), jnp.float32),
            jax.ShapeDtypeStruct((NPAD, 1), jnp.float32),
            jax.ShapeDtypeStruct((NPAD, 1), jnp.float32),
            jax.ShapeDtypeStruct((1, 128), jnp.float32),
        ],
    )(*ins)


def _make_sc_segment():
    mesh = plsc.VectorSubcoreMesh(core_axis_name="c", subcore_axis_name="s")

    @functools.partial(
        pl.kernel,
        mesh=mesh,
        out_type=jax.ShapeDtypeStruct((2, ACCR, DEH), jnp.float32),
        scratch_types=[
            pltpu.VMEM((NPAD,), jnp.float32),
            pltpu.VMEM((NPAD,), jnp.float32),
            pltpu.VMEM((128,), jnp.float32),
            pltpu.VMEM((CPT, CHUNK), jnp.int32),
            pltpu.VMEM((CHUNK,), jnp.int32),
            pltpu.VMEM((CHUNK,), jnp.int32),
            pltpu.VMEM((CHUNK,), jnp.float32),
            pltpu.VMEM((CHUNK, DEH), jnp.float32),
            pltpu.VMEM((CHUNK,), jnp.int32),
            pltpu.VMEM((CHUNK,), jnp.int32),
            pltpu.VMEM((CHUNK,), jnp.float32),
            pltpu.VMEM((CHUNK, DEH), jnp.float32),
            pltpu.VMEM((CHUNK,), jnp.int32),
            pltpu.VMEM((CHUNK,), jnp.int32),
            pltpu.VMEM((CHUNK,), jnp.float32),
            pltpu.VMEM((CHUNK, DEH), jnp.float32),
            pltpu.VMEM_SHARED((ACCR, DEH), jnp.float32),
            pltpu.SemaphoreType.DMA,
            pltpu.SemaphoreType.DMA,
            pltpu.SemaphoreType.DMA,
            pltpu.SemaphoreType.DMA,
            pltpu.SemaphoreType.DMA,
            pltpu.SemaphoreType.DMA,
        ],
        compiler_params=pltpu.CompilerParams(needs_layout_passes=False,
                                             use_tc_tiling_on_sc=False),
    )
    def seg(hh_hbm, es_hbm, ed_hbm, mx_hbm, eidx_hbm, out_hbm,
            es_v, ed_v, mx_v, eidx_v,
            s_a, d_a, w_a, r_a, s_b, d_b, w_b, r_b, s_c, d_c, w_c, r_c,
            acc_sh, gsem_a, gsem_b, gsem_c, asem_a, asem_b, asem_c):
        c = lax.axis_index("c")
        s = lax.axis_index("s")
        bufs = ((s_a, d_a, w_a, r_a, gsem_a, asem_a),
                (s_b, d_b, w_b, r_b, gsem_b, asem_b),
                (s_c, d_c, w_c, r_c, gsem_c, asem_c))

        pltpu.sync_copy(es_hbm, es_v)
        pltpu.sync_copy(ed_hbm, ed_v)
        pltpu.sync_copy(mx_hbm, mx_v)
        pltpu.sync_copy(eidx_hbm.at[s], eidx_v)
        m16 = mx_v[pl.ds(0, 16)]

        def _zrow(r, carry):
            for j in range(DEH // 16):
                r_a[r, pl.ds(16 * j, 16)] = jnp.zeros((16,), jnp.float32)
                r_b[r, pl.ds(16 * j, 16)] = jnp.zeros((16,), jnp.float32)
                r_c[r, pl.ds(16 * j, 16)] = jnp.zeros((16,), jnp.float32)
            return carry
        lax.fori_loop(0, CHUNK, _zrow, 0)
        for v in range(CHUNK // 16):
            d_a[pl.ds(16 * v, 16)] = jnp.zeros((16,), jnp.int32)
            d_b[pl.ds(16 * v, 16)] = jnp.zeros((16,), jnp.int32)
            d_c[pl.ds(16 * v, 16)] = jnp.zeros((16,), jnp.int32)
        for off, sz in ((0, 128), (128, 128), (256, 128), (384, 128),
                        (512, 120)):
            pltpu.sync_copy(r_a.at[pl.ds(0, sz)],
                            acc_sh.at[pl.ds(s * RPT + off, sz)])
        plsc.subcore_barrier()

        for s_x, d_x, w_x, r_x, gsem, asem in bufs:
            pltpu.async_copy(r_x, acc_sh.at[d_x], asem, add=True)

        def _weights(ci, s_x, d_x, w_x):
            def _wvec(v, c2):
                pk = eidx_v[ci, pl.ds(16 * v, 16)]
                si = jax.lax.shift_right_logical(pk, 14)
                di = jax.lax.bitwise_and(pk, PACK - 1)
                s_x[pl.ds(16 * v, 16)] = si
                d_x[pl.ds(16 * v, 16)] = di
                es_g = plsc.load_gather(es_v, [si])
                ed_g = plsc.load_gather(ed_v, [di])
                e = es_g + ed_g
                e = jnp.where(e > 0.0, e, 0.2 * e)
                w_x[pl.ds(16 * v, 16)] = jnp.exp(
                    e - jnp.maximum(ed_g + m16, 0.0))
                return c2
            lax.fori_loop(0, CHUNK // 16, _wvec, 0)

        def _lanebc(wv, l):
            return lax.gather(
                wv, jnp.full((16, 1), l, jnp.int32),
                lax.GatherDimensionNumbers(
                    offset_dims=(), collapsed_slice_dims=(0,),
                    start_index_map=(0,)),
                slice_sizes=(1,),
                mode=lax.GatherScatterMode.PROMISE_IN_BOUNDS)

        def _scale(r_x, w_x):
            def _svec(v, c2):
                wv = w_x[pl.ds(16 * v, 16)]
                for l in range(0, 16, 2):
                    r0 = 16 * v + l
                    r1 = r0 + 1
                    ws0 = _lanebc(wv, l)
                    ws1 = _lanebc(wv, l + 1)
                    t0 = [r_x[r0, pl.ds(16 * j, 16)] for j in range(DEH // 16)]
                    t1 = [r_x[r1, pl.ds(16 * j, 16)] for j in range(DEH // 16)]
                    for j in range(DEH // 16):
                        r_x[r0, pl.ds(16 * j, 16)] = t0[j] * ws0
                    for j in range(DEH // 16):
                        r_x[r1, pl.ds(16 * j, 16)] = t1[j] * ws1
                return c2
            lax.fori_loop(0, CHUNK // 16, _svec, 0)

        def _trip(ci3, carry):
            base = 3 * ci3
            gs = []
            for k in range(3):
                s_x, d_x, w_x, r_x, gsem, asem = bufs[k]
                pltpu.make_async_copy(r_x, acc_sh.at[d_x], asem).wait()
                _weights(base + k, s_x, d_x, w_x)
                gs.append(pltpu.async_copy(hh_hbm.at[c].at[s_x], r_x, gsem))
            for k in range(3):
                s_x, d_x, w_x, r_x, gsem, asem = bufs[k]
                gs[k].wait()
                _scale(r_x, w_x)
                pltpu.async_copy(r_x, acc_sh.at[d_x], asem, add=True)
            return carry
        lax.fori_loop(0, CPT // 3, _trip, 0)

        for s_x, d_x, w_x, r_x, gsem, asem in bufs:
            pltpu.make_async_copy(r_x, acc_sh.at[d_x], asem).wait()
        plsc.subcore_barrier()
        pltpu.sync_copy(acc_sh.at[pl.ds(s * RPT, RPT)],
                        out_hbm.at[c, pl.ds(s * RPT, RPT)])

    return seg


_sc_segment = _make_sc_segment()


def _segment_sc(hh, esf, edf, mx, eidx):
    return _sc_segment(hh, esf, edf, mx, eidx)


def kernel(x, edge_index, batch, W0, a_src0, a_dst0, b0,
           W1, a_src1, a_dst1, b1, Wp, bp):
    loop = jnp.arange(N, dtype=jnp.int32)
    src = jnp.concatenate([edge_index[0].astype(jnp.int32), loop])
    dst = jnp.concatenate([edge_index[1].astype(jnp.int32), loop])
    ne = src.shape[0]
    eidx = jnp.pad(src * PACK + dst, (0, ETOT - ne),
                   constant_values=PAD_DST).reshape(NSUB, CPT, CHUNK)

    hh0, es0, ed0, mx0 = _run_layer_tc(_layer0_tc, x, W0, a_src0, a_dst0)
    acc0 = _segment_sc(hh0, es0.reshape(NPAD), ed0.reshape(NPAD),
                       mx0.reshape(128), eidx)

    hh1, es1, ed1, mx1 = _run_layer_tc(_layer1_tc, acc0, W1, a_src1, a_dst1,
                                       extra_b=b0)
    acc1 = _segment_sc(hh1, es1.reshape(NPAD), ed1.reshape(NPAD),
                       mx1.reshape(128), eidx)

    node0 = jnp.searchsorted(batch, jnp.arange(BATCH_SIZE, dtype=batch.dtype)
                             ).astype(jnp.int32)
    h64 = pl.pallas_call(
        _head_gather_tc,
        grid_spec=pltpu.PrefetchScalarGridSpec(
            num_scalar_prefetch=1,
            grid=(BATCH_SIZE,),
            in_specs=[
                pl.BlockSpec((2, 1, 1, DEH), lambda j, n0: (0, n0[j], 0, 0)),
                pl.BlockSpec((1, 128), lambda j, n0: (0, 0)),
            ],
            out_specs=pl.BlockSpec((1, 1, 128), lambda j, n0: (j, 0, 0)),
        ),
        out_shape=jax.ShapeDtypeStruct((BATCH_SIZE, 1, 128), jnp.float32),
    )(node0, acc1.reshape(2, ACCR, 1, DEH), b1.reshape(1, 128))
    h64 = h64.reshape(BATCH_SIZE, 128)

    logits = pl.pallas_call(
        _head_mm_tc,
        in_specs=[
            pl.BlockSpec((BATCH_SIZE, 128), lambda: (0, 0)),
            pl.BlockSpec((128, NUM_CLASSES * MAX_SEQ_LEN), lambda: (0, 0)),
            pl.BlockSpec((1, NUM_CLASSES * MAX_SEQ_LEN), lambda: (0, 0)),
        ],
        out_specs=pl.BlockSpec((BATCH_SIZE, NUM_CLASSES * MAX_SEQ_LEN),
                               lambda: (0, 0)),
        out_shape=jax.ShapeDtypeStruct(
            (BATCH_SIZE, NUM_CLASSES * MAX_SEQ_LEN), jnp.float32),
    )(h64, Wp, bp.reshape(1, NUM_CLASSES * MAX_SEQ_LEN))

    return logits.reshape(MAX_SEQ_LEN * BATCH_SIZE, NUM_CLASSES)
